# baseline (speedup 1.0000x reference)
import functools

import jax
import jax.numpy as jnp
from jax import lax
from jax.experimental import pallas as pl
from jax.experimental.pallas import tpu as pltpu
from jax.experimental.pallas import tpu_sc as plsc


def _tr_body(E, BLKV, in_ref, out_ref):
    sub = BLKV // 4
    x = in_ref[0]
    parts = []
    for a in range(4):
        sel = jnp.pad(jnp.eye(E, dtype=jnp.float32),
                      ((0, 0), (a * E, 128 - (a + 1) * E)))
        parts.append(jax.lax.dot_general(x[:, a * sub:(a + 1) * sub], sel,
                                         (((0,), (0,)), ((), ())),
                                         preferred_element_type=jnp.float32))
    acc = (parts[0] + parts[1]) + (parts[2] + parts[3])
    out_ref[0] = acc.astype(jnp.bfloat16)


def _make_linearize(N_CAT, V, E, BLKV):
    n_j = (V + BLKV - 1) // BLKV
    return pl.pallas_call(
        functools.partial(_tr_body, E, BLKV),
        grid=(N_CAT, n_j),
        in_specs=[pl.BlockSpec((1, E, BLKV), lambda c, j: (c, 0, j))],
        out_specs=pl.BlockSpec((1, BLKV // 4, 128), lambda c, j: (c, j, 0)),
        out_shape=jax.ShapeDtypeStruct((N_CAT, n_j * BLKV // 4, 128),
                                       jnp.bfloat16),
    )


def _make_sc_gather(R, E, CH):
    info = plsc.get_sparse_core_info()
    NC, NS = info.num_cores, info.num_subcores
    NW = NC * NS
    rows_per_w = R // NW
    n_ch = rows_per_w // CH

    mesh = plsc.VectorSubcoreMesh(core_axis_name="c", subcore_axis_name="s")

    @functools.partial(
        pl.kernel,
        mesh=mesh,
        compiler_params=pltpu.CompilerParams(use_tc_tiling_on_sc=False),
        out_type=jax.ShapeDtypeStruct((R, E), jnp.bfloat16),
        scratch_types=[
            pltpu.VMEM((rows_per_w,), jnp.int32),
            pltpu.VMEM((CH, E), jnp.bfloat16),
            pltpu.VMEM((CH, E), jnp.bfloat16),
            pltpu.SemaphoreType.DMA,
            pltpu.SemaphoreType.DMA,
        ],
    )
    def sc_gather(tab_hbm, idx_hbm, out_hbm, idx_v, buf0, buf1, sem0, sem1):
        wid = lax.axis_index("s") * NC + lax.axis_index("c")
        base = wid * rows_per_w
        pltpu.sync_copy(idx_hbm.at[pl.ds(base, rows_per_w)], idx_v)
        bufs = (buf0, buf1)
        sems = (sem0, sem1)
        cps = [None, None]
        for k in range(n_ch):
            cps[k % 2] = pltpu.async_copy(
                tab_hbm.at[idx_v.at[pl.ds(k * CH, CH)]], bufs[k % 2], sems[k % 2])
            if k > 0:
                cps[(k - 1) % 2].wait()
                pltpu.sync_copy(bufs[(k - 1) % 2],
                                out_hbm.at[pl.ds(base + (k - 1) * CH, CH)])
        cps[(n_ch - 1) % 2].wait()
        pltpu.sync_copy(bufs[(n_ch - 1) % 2],
                        out_hbm.at[pl.ds(base + (n_ch - 1) * CH, CH)])

    return sc_gather


def _stats_body(nb, num_ref, emb_ref, onum_ref, oemb_ref, snum, semb):
    i = pl.program_id(0)

    @pl.when(i == 0)
    def _():
        snum[...] = jnp.zeros_like(snum)
        semb[...] = jnp.zeros_like(semb)

    num = num_ref[...]
    emb = emb_ref[...].astype(jnp.float32)
    snum[0:1, :] += jnp.sum(num, axis=0, keepdims=True)
    snum[1:2, :] += jnp.sum(num * num, axis=0, keepdims=True)
    semb[0:1, :] += jnp.sum(emb, axis=0, keepdims=True)
    semb[1:2, :] += jnp.sum(emb * emb, axis=0, keepdims=True)

    @pl.when(i == nb - 1)
    def _():
        onum_ref[...] = snum[...]
        oemb_ref[...] = semb[...]


def _make_stats(B, N_NUM, F_EMB, BLK):
    nb = B // BLK
    return pl.pallas_call(
        functools.partial(_stats_body, nb),
        grid=(nb,),
        in_specs=[
            pl.BlockSpec((BLK, N_NUM), lambda i: (i, 0)),
            pl.BlockSpec((BLK, F_EMB), lambda i: (i, 0)),
        ],
        out_specs=[
            pl.BlockSpec((2, N_NUM), lambda i: (0, 0)),
            pl.BlockSpec((2, F_EMB), lambda i: (0, 0)),
        ],
        out_shape=[
            jax.ShapeDtypeStruct((2, N_NUM), jnp.float32),
            jax.ShapeDtypeStruct((2, F_EMB), jnp.float32),
        ],
        scratch_shapes=[
            pltpu.VMEM((2, N_NUM), jnp.float32),
            pltpu.VMEM((2, F_EMB), jnp.float32),
        ],
    )


def _norm_body(num_ref, emb_ref, ssnum_ref, ssemb_ref, out_ref):
    n = num_ref[...] * ssnum_ref[0:1, :] + ssnum_ref[1:2, :]
    e = emb_ref[...].astype(jnp.float32) * ssemb_ref[0:1, :] + ssemb_ref[1:2, :]
    out_ref[...] = jnp.concatenate([n, e], axis=1)


def _make_norm(B, N_NUM, F_EMB, BLK):
    nb = B // BLK
    return pl.pallas_call(
        _norm_body,
        grid=(nb,),
        in_specs=[
            pl.BlockSpec((BLK, N_NUM), lambda i: (i, 0)),
            pl.BlockSpec((BLK, F_EMB), lambda i: (i, 0)),
            pl.BlockSpec((2, N_NUM), lambda i: (0, 0)),
            pl.BlockSpec((2, F_EMB), lambda i: (0, 0)),
        ],
        out_specs=pl.BlockSpec((BLK, N_NUM + F_EMB), lambda i: (i, 0)),
        out_shape=jax.ShapeDtypeStruct((B, N_NUM + F_EMB), jnp.float32),
    )


def kernel(numeric, cat_idx, tables, gamma, beta):
    B, N_NUM = numeric.shape
    N_CAT, V, E = tables.shape
    F_EMB = N_CAT * E
    R = B * N_CAT
    CH = 1024

    tt = jnp.transpose(tables, (0, 2, 1))
    BLKV = 8192
    SUB = BLKV // 4
    n_j = (V + BLKV - 1) // BLKV
    VPAD = n_j * SUB
    lin = _make_linearize(N_CAT, V, E, BLKV)(tt)
    tab_flat = lin.reshape(N_CAT * VPAD * 4, E)
    del tables
    v = cat_idx.astype(jnp.int32)
    c = jnp.arange(N_CAT, dtype=jnp.int32)[None, :]
    rem = v % BLKV
    rho = ((c * VPAD + (v // BLKV) * SUB + rem % SUB) * 4) + rem // SUB
    flat_idx = rho.reshape(R)

    emb_rows = _make_sc_gather(R, E, CH)(tab_flat, flat_idx)
    emb = emb_rows.reshape(B, F_EMB)

    sums_num, sums_emb = _make_stats(B, N_NUM, F_EMB, 1024)(numeric, emb)
    sums = jnp.concatenate([sums_num, sums_emb], axis=1)
    mean = sums[0] / B
    var = sums[1] / B - mean * mean
    scale = gamma * lax.rsqrt(var + 1e-5)
    shift = beta - mean * scale
    ss = jnp.stack([scale, shift])

    return _make_norm(B, N_NUM, F_EMB, 1024)(
        numeric, emb, ss[:, :N_NUM], ss[:, N_NUM:])


# speedup vs baseline: 2.1271x; 2.1271x over previous
import functools

import jax
import jax.numpy as jnp
from jax import lax
from jax.experimental import pallas as pl
from jax.experimental.pallas import tpu as pltpu
from jax.experimental.pallas import tpu_sc as plsc


def _tr_body(E, BLKV, in_ref, out_ref):
    sub = BLKV // 4
    x = in_ref[0]
    xs = jnp.concatenate([x[:, a * sub:(a + 1) * sub] for a in range(4)],
                         axis=0)
    out_ref[0] = jax.lax.dot_general(xs, jnp.eye(128, dtype=jnp.float32),
                                     (((0,), (0,)), ((), ())),
                                     preferred_element_type=jnp.float32)


def _make_linearize(N_CAT, V, E, BLKV):
    n_j = (V + BLKV - 1) // BLKV
    return pl.pallas_call(
        functools.partial(_tr_body, E, BLKV),
        grid=(N_CAT, n_j),
        in_specs=[pl.BlockSpec((1, E, BLKV), lambda c, j: (c, 0, j))],
        out_specs=pl.BlockSpec((1, BLKV // 4, 128), lambda c, j: (c, j, 0)),
        out_shape=jax.ShapeDtypeStruct((N_CAT, n_j * BLKV // 4, 128),
                                       jnp.float32),
    )


def _make_sc_gather(R, E, CH):
    info = plsc.get_sparse_core_info()
    NC, NS = info.num_cores, info.num_subcores
    NW = NC * NS
    rows_per_w = R // NW
    n_ch = rows_per_w // CH

    mesh = plsc.VectorSubcoreMesh(core_axis_name="c", subcore_axis_name="s")

    @functools.partial(
        pl.kernel,
        mesh=mesh,
        compiler_params=pltpu.CompilerParams(use_tc_tiling_on_sc=False),
        out_type=jax.ShapeDtypeStruct((R, E), jnp.float32),
        scratch_types=[
            pltpu.VMEM((rows_per_w,), jnp.int32),
            pltpu.VMEM((CH, E), jnp.float32),
            pltpu.VMEM((CH, E), jnp.float32),
            pltpu.SemaphoreType.DMA,
            pltpu.SemaphoreType.DMA,
        ],
    )
    def sc_gather(tab_hbm, idx_hbm, out_hbm, idx_v, buf0, buf1, sem0, sem1):
        wid = lax.axis_index("s") * NC + lax.axis_index("c")
        base = wid * rows_per_w
        pltpu.sync_copy(idx_hbm.at[pl.ds(base, rows_per_w)], idx_v)
        bufs = (buf0, buf1)
        sems = (sem0, sem1)
        cps = [None, None]
        for k in range(n_ch):
            cps[k % 2] = pltpu.async_copy(
                tab_hbm.at[idx_v.at[pl.ds(k * CH, CH)]], bufs[k % 2], sems[k % 2])
            if k > 0:
                cps[(k - 1) % 2].wait()
                pltpu.sync_copy(bufs[(k - 1) % 2],
                                out_hbm.at[pl.ds(base + (k - 1) * CH, CH)])
        cps[(n_ch - 1) % 2].wait()
        pltpu.sync_copy(bufs[(n_ch - 1) % 2],
                        out_hbm.at[pl.ds(base + (n_ch - 1) * CH, CH)])

    return sc_gather


def _bn_body(B, num_ref, emb_ref, gbn_ref, gbe_ref, out_ref,
             snum, semb, ssnum, ssemb):
    p = pl.program_id(0)
    i = pl.program_id(1)

    @pl.when((p == 0) & (i == 0))
    def _():
        snum[...] = jnp.zeros_like(snum)
        semb[...] = jnp.zeros_like(semb)

    @pl.when(p == 0)
    def _():
        num = num_ref[...]
        emb = emb_ref[...]
        snum[0:1, :] += jnp.sum(num, axis=0, keepdims=True)
        snum[1:2, :] += jnp.sum(num * num, axis=0, keepdims=True)
        semb[0:1, :] += jnp.sum(emb, axis=0, keepdims=True)
        semb[1:2, :] += jnp.sum(emb * emb, axis=0, keepdims=True)

    @pl.when((p == 1) & (i == 0))
    def _():
        for s, gb, ss in ((snum, gbn_ref, ssnum), (semb, gbe_ref, ssemb)):
            mean = s[0:1, :] / B
            var = s[1:2, :] / B - mean * mean
            scale = gb[0:1, :] * lax.rsqrt(var + 1e-5)
            ss[0:1, :] = scale
            ss[1:2, :] = gb[1:2, :] - mean * scale

    @pl.when(p == 1)
    def _():
        n = num_ref[...] * ssnum[0:1, :] + ssnum[1:2, :]
        e = emb_ref[...] * ssemb[0:1, :] + ssemb[1:2, :]
        out_ref[...] = jnp.concatenate([n, e], axis=1)


def _make_bn(B, N_NUM, F_EMB, BLK):
    nb = B // BLK
    return pl.pallas_call(
        functools.partial(_bn_body, B),
        grid=(2, nb),
        in_specs=[
            pl.BlockSpec((BLK, N_NUM), lambda p, i: (i, 0)),
            pl.BlockSpec((BLK, F_EMB), lambda p, i: (i, 0)),
            pl.BlockSpec((2, N_NUM), lambda p, i: (0, 0)),
            pl.BlockSpec((2, F_EMB), lambda p, i: (0, 0)),
        ],
        out_specs=pl.BlockSpec((BLK, N_NUM + F_EMB),
                               lambda p, i: (jnp.where(p == 0, 0, i), 0)),
        out_shape=jax.ShapeDtypeStruct((B, N_NUM + F_EMB), jnp.float32),
        scratch_shapes=[
            pltpu.VMEM((2, N_NUM), jnp.float32),
            pltpu.VMEM((2, F_EMB), jnp.float32),
            pltpu.VMEM((2, N_NUM), jnp.float32),
            pltpu.VMEM((2, F_EMB), jnp.float32),
        ],
    )


def kernel(numeric, cat_idx, tables, gamma, beta):
    B, N_NUM = numeric.shape
    N_CAT, V, E = tables.shape
    F_EMB = N_CAT * E
    R = B * N_CAT
    CH = 1024

    tt = jnp.transpose(tables, (0, 2, 1))
    BLKV = 8192
    SUB = BLKV // 4
    n_j = (V + BLKV - 1) // BLKV
    VPAD = n_j * SUB
    lin = _make_linearize(N_CAT, V, E, BLKV)(tt)
    tab_flat = lin.reshape(N_CAT * VPAD * 4, E)
    del tables
    v = cat_idx.astype(jnp.int32)
    c = jnp.arange(N_CAT, dtype=jnp.int32)[None, :]
    rem = v % BLKV
    rho = ((c * VPAD + (v // BLKV) * SUB + rem % SUB) * 4) + rem // SUB
    flat_idx = rho.reshape(R)

    emb_rows = _make_sc_gather(R, E, CH)(tab_flat, flat_idx)
    emb = emb_rows.reshape(B, F_EMB)

    gb = jnp.stack([gamma, beta])
    return _make_bn(B, N_NUM, F_EMB, 1024)(
        numeric, emb, gb[:, :N_NUM], gb[:, N_NUM:])
